# Initial kernel scaffold; baseline (speedup 1.0000x reference)
import jax
import jax.numpy as jnp
from jax.experimental import pallas as pl


def kernel(x, edge_index, edge_attr, state, batch, params):
    raise NotImplementedError("write your pallas kernel here")


# speedup vs baseline: 4.1140x; 4.1140x over previous
import functools

import jax
import jax.numpy as jnp
from jax import lax
from jax.experimental import pallas as pl
from jax.experimental.pallas import tpu as pltpu
from jax.experimental.pallas import tpu_sc as plsc

N_NODES = 10000
N_EDGES = 320000
N_GRAPH = 64
NPAD = 10240
NC, NS = 2, 16
NW = NC * NS
EPW = N_EDGES // NW
CH = 80
NCH = EPW // CH
SL = NPAD // NS

_BLK = 4000
_GE = N_EDGES // _BLK
_BLK6 = 8000
_G6 = (N_EDGES * 32) // 128 // _BLK6


def _acc_stats(st_ref, h):
    s = jnp.sum(h, axis=0, keepdims=True)
    ss = jnp.sum(h * h, axis=0, keepdims=True)
    part = jnp.concatenate(
        [s, ss, jnp.zeros((6, h.shape[1]), jnp.float32)], axis=0)
    i = pl.program_id(0)

    @pl.when(i == 0)
    def _():
        st_ref[...] = part

    @pl.when(i > 0)
    def _():
        st_ref[...] = st_ref[...] + part


def _dot(a, b):
    return jnp.dot(a, b, preferred_element_type=jnp.float32)


def _bn_full(h, g, be):
    m = jnp.mean(h, axis=0, keepdims=True)
    var = jnp.mean(h * h, axis=0, keepdims=True) - m * m
    sc = g / jnp.sqrt(var + 1e-5)
    return h * sc + (be - m * sc)


def _cspec(r, c):
    return pl.BlockSpec((r, c), lambda i: (0, 0))


def _pe1_body(ea, w0, b0, st):
    h0 = jnp.maximum(_dot(ea[...], w0[...]) + b0[...], 0.0)
    _acc_stats(st, h0)


_pe1_call = pl.pallas_call(
    _pe1_body,
    grid=(_GE,),
    in_specs=[pl.BlockSpec((_BLK, 16), lambda i: (i, 0)),
              _cspec(16, 64), _cspec(1, 64)],
    out_specs=_cspec(8, 64),
    out_shape=jax.ShapeDtypeStruct((8, 64), jnp.float32),
)


def _pe2_body(ea, w0, b0, w1, b1, h1o, st):
    h0 = jnp.maximum(_dot(ea[...], w0[...]) + b0[...], 0.0)
    h1 = jnp.maximum(_dot(h0, w1[...]) + b1[...], 0.0)
    h1o[...] = h1
    _acc_stats(st, h1)


_pe2_call = pl.pallas_call(
    _pe2_body,
    grid=(_GE,),
    in_specs=[pl.BlockSpec((_BLK, 16), lambda i: (i, 0)),
              _cspec(16, 64), _cspec(1, 64),
              _cspec(64, 32), _cspec(1, 32)],
    out_specs=[pl.BlockSpec((_BLK, 32), lambda i: (i, 0)), _cspec(8, 32)],
    out_shape=[jax.ShapeDtypeStruct((N_EDGES, 32), jnp.float32),
               jax.ShapeDtypeStruct((8, 32), jnp.float32)],
)


def _pe3_body(h1, ts, td, w, b, g0o, st):
    g0 = jnp.maximum(_dot(h1[...], w[...]) + ts[...] + td[...] + b[...], 0.0)
    g0o[...] = g0
    _acc_stats(st, g0)


_pe3_call = pl.pallas_call(
    _pe3_body,
    grid=(_GE,),
    in_specs=[pl.BlockSpec((_BLK, 32), lambda i: (i, 0)),
              pl.BlockSpec((_BLK, 32), lambda i: (i, 0)),
              pl.BlockSpec((_BLK, 32), lambda i: (i, 0)),
              _cspec(32, 32), _cspec(1, 32)],
    out_specs=[pl.BlockSpec((_BLK, 32), lambda i: (i, 0)), _cspec(8, 32)],
    out_shape=[jax.ShapeDtypeStruct((N_EDGES, 32), jnp.float32),
               jax.ShapeDtypeStruct((8, 32), jnp.float32)],
)


def _mk_lin32(relu):
    def body(xin, w, b, yo, st):
        y = _dot(xin[...], w[...]) + b[...]
        if relu:
            y = jnp.maximum(y, 0.0)
        yo[...] = y
        _acc_stats(st, y)

    return pl.pallas_call(
        body,
        grid=(_GE,),
        in_specs=[pl.BlockSpec((_BLK, 32), lambda i: (i, 0)),
                  _cspec(32, 32), _cspec(1, 32)],
        out_specs=[pl.BlockSpec((_BLK, 32), lambda i: (i, 0)), _cspec(8, 32)],
        out_shape=[jax.ShapeDtypeStruct((N_EDGES, 32), jnp.float32),
                   jax.ShapeDtypeStruct((8, 32), jnp.float32)],
    )


_pe4_call = _mk_lin32(True)
_pe5_call = _mk_lin32(False)


def _pe6_body(g2r, h1r, a4, a1, cc, e1o):
    e1o[...] = g2r[...] * a4[...] + h1r[...] * a1[...] + cc[...]


_pe6_call = pl.pallas_call(
    _pe6_body,
    grid=(_G6,),
    in_specs=[pl.BlockSpec((_BLK6, 128), lambda i: (i, 0)),
              pl.BlockSpec((_BLK6, 128), lambda i: (i, 0)),
              _cspec(1, 128), _cspec(1, 128), _cspec(1, 128)],
    out_specs=pl.BlockSpec((_BLK6, 128), lambda i: (i, 0)),
    out_shape=jax.ShapeDtypeStruct((N_EDGES * 32 // 128, 128), jnp.float32),
)


def _pu_body(stt, w0, b0, g0, be0, w1, b1, g1, be1, dT, uo, ro):
    h = jnp.maximum(_dot(stt[...], w0[...]) + b0[...], 0.0)
    h = _bn_full(h, g0[...], be0[...])
    h = jnp.maximum(_dot(h, w1[...]) + b1[...], 0.0)
    u = _bn_full(h, g1[...], be1[...])
    uo[...] = u
    ro[...] = _dot(u, dT[...])


_pu_call = pl.pallas_call(
    _pu_body,
    out_shape=[jax.ShapeDtypeStruct((N_GRAPH, 32), jnp.float32),
               jax.ShapeDtypeStruct((N_GRAPH, 32), jnp.float32)],
)


def _pv_body(xx, bf, rtab, w0, b0, g0, be0, w1, b1, g1, be1, aT, bmT,
             vo, po, qo):
    h = jnp.maximum(_dot(xx[...], w0[...]) + b0[...], 0.0)
    h = _bn_full(h, g0[...], be0[...])
    h = jnp.maximum(_dot(h, w1[...]) + b1[...], 0.0)
    v = _bn_full(h, g1[...], be1[...])
    vo[...] = v
    oh = (bf[...] == lax.broadcasted_iota(
        jnp.int32, (N_GRAPH, N_NODES), 0).astype(jnp.float32)
          ).astype(jnp.float32)
    rn = lax.dot_general(oh, rtab[...], (((0,), (0,)), ((), ())),
                         preferred_element_type=jnp.float32)
    po[...] = _dot(v, aT[...]) + rn
    qo[...] = _dot(v, bmT[...])


_pv_call = pl.pallas_call(
    _pv_body,
    out_shape=[jax.ShapeDtypeStruct((N_NODES, 32), jnp.float32),
               jax.ShapeDtypeStruct((N_NODES, 32), jnp.float32),
               jax.ShapeDtypeStruct((N_NODES, 32), jnp.float32)],
)


def _pn_body2(v_, accp, cntp, u_, bf,
              wa, wb, wc, b0, g0, be0, w1, b1, g1, be1, w2, b2, g2, be2,
              sa, sb, sc_, c0, f0, fb0, c1, cb1, f1, fb1, c2, cb2, f2, fb2,
              v1o, u1o):
    acc = accp[0, :N_NODES, :] + accp[1, :N_NODES, :]
    cnt = cntp[0, :N_NODES, :] + cntp[1, :N_NODES, :]
    v = v_[...]
    u = u_[...]
    v_mean = acc / jnp.maximum(cnt, 1.0)
    oh = (bf[...] == lax.broadcasted_iota(
        jnp.int32, (N_GRAPH, N_NODES), 0).astype(jnp.float32)
          ).astype(jnp.float32)
    u_bn = lax.dot_general(oh, u, (((0,), (0,)), ((), ())),
                           preferred_element_type=jnp.float32)
    m = jnp.maximum(
        _dot(v, wa[...]) + _dot(v_mean, wb[...]) + _dot(u_bn, wc[...])
        + b0[...], 0.0)
    m = _bn_full(m, g0[...], be0[...])
    m = jnp.maximum(_dot(m, w1[...]) + b1[...], 0.0)
    m = _bn_full(m, g1[...], be1[...])
    m = _dot(m, w2[...]) + b2[...]
    v1 = _bn_full(m, g2[...], be2[...]) + v
    v1o[...] = v1
    u_e = _dot(oh, acc) / jnp.maximum(_dot(oh, cnt), 1.0)
    cntb = jnp.sum(oh, axis=1, keepdims=True)
    u_v = _dot(oh, v1) / jnp.maximum(cntb, 1.0)
    m = jnp.maximum(
        _dot(u_e, sa[...]) + _dot(u_v, sb[...]) + _dot(u, sc_[...])
        + c0[...], 0.0)
    m = _bn_full(m, f0[...], fb0[...])
    m = jnp.maximum(_dot(m, c1[...]) + cb1[...], 0.0)
    m = _bn_full(m, f1[...], fb1[...])
    m = _dot(m, c2[...]) + cb2[...]
    u1o[...] = _bn_full(m, f2[...], fb2[...]) + u


_pn_call = pl.pallas_call(
    _pn_body2,
    out_shape=[jax.ShapeDtypeStruct((N_NODES, 32), jnp.float32),
               jax.ShapeDtypeStruct((N_GRAPH, 32), jnp.float32)],
)


_SC_CACHE = {}


def _sc_kernels():
    if _SC_CACHE:
        return _SC_CACHE['gather'], _SC_CACHE['scatter']

    mesh = plsc.VectorSubcoreMesh(core_axis_name="c", subcore_axis_name="s",
                                  num_cores=NC, num_subcores=NS)

    @functools.partial(
        pl.kernel,
        mesh=mesh,
        out_type=[jax.ShapeDtypeStruct((N_EDGES, 32), jnp.float32),
                  jax.ShapeDtypeStruct((N_EDGES, 32), jnp.float32)],
        compiler_params=pltpu.CompilerParams(use_tc_tiling_on_sc=False),
        scratch_types=[pltpu.VMEM((CH,), jnp.int32),
                       pltpu.VMEM((CH,), jnp.int32),
                       pltpu.VMEM((CH, 32), jnp.float32),
                       pltpu.VMEM((CH, 32), jnp.float32),
                       pltpu.SemaphoreType.DMA,
                       pltpu.SemaphoreType.DMA],
    )
    def sc_gather(pp_hbm, q_hbm, src_hbm, dst_hbm, ts_hbm, td_hbm,
                  si_v, di_v, rs_v, rd_v, sem1, sem2):
        c = lax.axis_index("c")
        s = lax.axis_index("s")
        base = (c * NS + s) * EPW

        def body(k, carry):
            off = base + k * CH
            pltpu.sync_copy(src_hbm.at[pl.ds(off, CH)], si_v)
            pltpu.sync_copy(dst_hbm.at[pl.ds(off, CH)], di_v)
            cp1 = pltpu.async_copy(pp_hbm.at[si_v], rs_v, sem1)
            cp2 = pltpu.async_copy(q_hbm.at[di_v], rd_v, sem2)
            cp1.wait()
            cp2.wait()
            pltpu.sync_copy(rs_v, ts_hbm.at[pl.ds(off, CH)])
            pltpu.sync_copy(rd_v, td_hbm.at[pl.ds(off, CH)])
            return carry

        lax.fori_loop(0, NCH, body, 0)

    @functools.partial(
        pl.kernel,
        mesh=mesh,
        out_type=[jax.ShapeDtypeStruct((NC, NPAD, 32), jnp.float32),
                  jax.ShapeDtypeStruct((NC, NPAD, 32), jnp.float32)],
        compiler_params=pltpu.CompilerParams(use_tc_tiling_on_sc=False),
        scratch_types=[pltpu.VMEM((CH,), jnp.int32),
                       pltpu.VMEM((CH, 32), jnp.float32),
                       pltpu.VMEM((CH, 32), jnp.float32),
                       pltpu.VMEM((SL, 32), jnp.float32),
                       pltpu.VMEM_SHARED((NPAD, 32), jnp.float32),
                       pltpu.VMEM_SHARED((NPAD, 32), jnp.float32),
                       pltpu.SemaphoreType.DMA],
    )
    def sc_scatter(e1_hbm, src_hbm, zeros_hbm, ones_hbm, acc_hbm, cnt_hbm,
                   idx_v, rows_v, ones_v, stage_v, accS, cntS, sem):
        c = lax.axis_index("c")
        s = lax.axis_index("s")
        base = (c * NS + s) * EPW
        pltpu.sync_copy(zeros_hbm, stage_v)
        pltpu.sync_copy(stage_v, accS.at[pl.ds(s * SL, SL)])
        pltpu.sync_copy(stage_v, cntS.at[pl.ds(s * SL, SL)])
        pltpu.sync_copy(ones_hbm, ones_v)
        plsc.subcore_barrier()

        def body(k, carry):
            off = base + k * CH
            pltpu.sync_copy(src_hbm.at[pl.ds(off, CH)], idx_v)
            pltpu.sync_copy(e1_hbm.at[pl.ds(off, CH)], rows_v)
            pltpu.sync_copy(rows_v, accS.at[idx_v], add=True)
            pltpu.sync_copy(ones_v, cntS.at[idx_v], add=True)
            return carry

        lax.fori_loop(0, NCH, body, 0)

        plsc.subcore_barrier()
        pltpu.sync_copy(accS.at[pl.ds(s * SL, SL)],
                        acc_hbm.at[c, pl.ds(s * SL, SL)])
        pltpu.sync_copy(cntS.at[pl.ds(s * SL, SL)],
                        cnt_hbm.at[c, pl.ds(s * SL, SL)])

    _SC_CACHE['gather'] = sc_gather
    _SC_CACHE['scatter'] = sc_scatter
    return sc_gather, sc_scatter


def _fold(st, n, g, be):
    m = st[0] / n
    var = st[1] / n - m * m
    sc = g / jnp.sqrt(var + 1e-5)
    sh = be - m * sc
    return sc, sh


@jax.jit
def kernel(x, edge_index, edge_attr, state, batch, params):
    src = edge_index[0].astype(jnp.int32)
    dst = edge_index[1].astype(jnp.int32)
    batchf = batch.astype(jnp.float32)[None, :]
    pe, pv, pu = params['e'], params['v'], params['u']
    pue, pun, pus = params['ue'], params['un'], params['us']
    w0ue = pue['W0']
    A, Bm, C, D = (w0ue[:, 0:32], w0ue[:, 32:64],
                   w0ue[:, 64:96], w0ue[:, 96:128])

    u, rtab = _pu_call(
        state, pu['W0'].T, pu['b0'][None], pu['g0'][None], pu['be0'][None],
        pu['W1'].T, pu['b1'][None], pu['g1'][None], pu['be1'][None], D.T)

    v, pp, q = _pv_call(
        x, batchf, rtab,
        pv['W0'].T, pv['b0'][None], pv['g0'][None], pv['be0'][None],
        pv['W1'].T, pv['b1'][None], pv['g1'][None], pv['be1'][None],
        A.T, Bm.T)

    nE = jnp.float32(N_EDGES)
    st0 = _pe1_call(edge_attr, pe['W0'].T, pe['b0'][None])
    s0, t0 = _fold(st0, nE, pe['g0'], pe['be0'])
    w1fT = (pe['W1'] * s0[None, :]).T
    b1f = (pe['b1'] + pe['W1'] @ t0)[None]
    h1, st1 = _pe2_call(edge_attr, pe['W0'].T, pe['b0'][None], w1fT, b1f)
    s1, t1 = _fold(st1, nE, pe['g1'], pe['be1'])

    sc_gather, sc_scatter = _sc_kernels()
    ts, td = sc_gather(pp, q, src, dst)

    c2T = (C * s1[None, :]).T
    c3 = (pue['b0'] + C @ t1)[None]
    g0, st2 = _pe3_call(h1, ts, td, c2T, c3)
    s2, t2 = _fold(st2, nE, pue['g0'], pue['be0'])

    w1f2T = (pue['W1'] * s2[None, :]).T
    b1f2 = (pue['b1'] + pue['W1'] @ t2)[None]
    g1, st3 = _pe4_call(g0, w1f2T, b1f2)
    s3, t3 = _fold(st3, nE, pue['g1'], pue['be1'])

    w2fT = (pue['W2'] * s3[None, :]).T
    b2f = (pue['b2'] + pue['W2'] @ t3)[None]
    g2, st4 = _pe5_call(g1, w2fT, b2f)
    s4, t4 = _fold(st4, nE, pue['g2'], pue['be2'])

    rr = N_EDGES * 32 // 128
    a4 = jnp.tile(s4, 4)[None]
    a1 = jnp.tile(s1, 4)[None]
    cc = jnp.tile(t4 + t1, 4)[None]
    e1r = _pe6_call(g2.reshape(rr, 128), h1.reshape(rr, 128), a4, a1, cc)
    e1 = e1r.reshape(N_EDGES, 32)

    zeros = jnp.zeros((SL, 32), jnp.float32)
    ones = jnp.ones((CH, 32), jnp.float32)
    accp, cntp = sc_scatter(e1, src, zeros, ones)

    v1, u1 = _pn_call(
        v, accp, cntp, u, batchf,
        pun['W0'][:, 0:32].T, pun['W0'][:, 32:64].T, pun['W0'][:, 64:96].T,
        pun['b0'][None], pun['g0'][None], pun['be0'][None],
        pun['W1'].T, pun['b1'][None], pun['g1'][None], pun['be1'][None],
        pun['W2'].T, pun['b2'][None], pun['g2'][None], pun['be2'][None],
        pus['W0'][:, 0:32].T, pus['W0'][:, 32:64].T, pus['W0'][:, 64:96].T,
        pus['b0'][None], pus['g0'][None], pus['be0'][None],
        pus['W1'].T, pus['b1'][None], pus['g1'][None], pus['be1'][None],
        pus['W2'].T, pus['b2'][None], pus['g2'][None], pus['be2'][None])

    return (v1, e1, u1)


# speedup vs baseline: 4.3105x; 1.0478x over previous
import functools

import jax
import jax.numpy as jnp
from jax import lax
from jax.experimental import pallas as pl
from jax.experimental.pallas import tpu as pltpu
from jax.experimental.pallas import tpu_sc as plsc

N_NODES = 10000
N_EDGES = 320000
N_GRAPH = 64
NPAD = 10240
NC, NS = 2, 16
NW = NC * NS
EPW = N_EDGES // NW
CH = 80
NCH = EPW // CH
SL = NPAD // NS
SCK = 2000
KSUB = SCK // CH
NSCK = EPW // SCK

_BLK = 4000
_GE = N_EDGES // _BLK
_BLK6 = 8000
_G6 = (N_EDGES * 32) // 128 // _BLK6


def _acc_stats(st_ref, h):
    s = jnp.sum(h, axis=0, keepdims=True)
    ss = jnp.sum(h * h, axis=0, keepdims=True)
    part = jnp.concatenate(
        [s, ss, jnp.zeros((6, h.shape[1]), jnp.float32)], axis=0)
    i = pl.program_id(0)

    @pl.when(i == 0)
    def _():
        st_ref[...] = part

    @pl.when(i > 0)
    def _():
        st_ref[...] = st_ref[...] + part


def _dot(a, b):
    return jnp.dot(a, b, preferred_element_type=jnp.float32)


def _bn_full(h, g, be):
    m = jnp.mean(h, axis=0, keepdims=True)
    var = jnp.mean(h * h, axis=0, keepdims=True) - m * m
    sc = g / jnp.sqrt(var + 1e-5)
    return h * sc + (be - m * sc)


def _cspec(r, c):
    return pl.BlockSpec((r, c), lambda i: (0, 0))


def _pe1_body(ea, w0, b0, st):
    h0 = jnp.maximum(_dot(ea[...], w0[...]) + b0[...], 0.0)
    _acc_stats(st, h0)


_pe1_call = pl.pallas_call(
    _pe1_body,
    grid=(_GE,),
    in_specs=[pl.BlockSpec((_BLK, 16), lambda i: (i, 0)),
              _cspec(16, 64), _cspec(1, 64)],
    out_specs=_cspec(8, 64),
    out_shape=jax.ShapeDtypeStruct((8, 64), jnp.float32),
)


def _pe2_body(ea, w0, b0, w1, b1, h1o, st):
    h0 = jnp.maximum(_dot(ea[...], w0[...]) + b0[...], 0.0)
    h1 = jnp.maximum(_dot(h0, w1[...]) + b1[...], 0.0)
    h1o[...] = h1
    _acc_stats(st, h1)


_pe2_call = pl.pallas_call(
    _pe2_body,
    grid=(_GE,),
    in_specs=[pl.BlockSpec((_BLK, 16), lambda i: (i, 0)),
              _cspec(16, 64), _cspec(1, 64),
              _cspec(64, 32), _cspec(1, 32)],
    out_specs=[pl.BlockSpec((_BLK, 32), lambda i: (i, 0)), _cspec(8, 32)],
    out_shape=[jax.ShapeDtypeStruct((N_EDGES, 32), jnp.float32),
               jax.ShapeDtypeStruct((8, 32), jnp.float32)],
)


def _pe3_body(h1, ts, td, w, b, g0o, st):
    g0 = jnp.maximum(_dot(h1[...], w[...]) + ts[...] + td[...] + b[...], 0.0)
    g0o[...] = g0
    _acc_stats(st, g0)


_pe3_call = pl.pallas_call(
    _pe3_body,
    grid=(_GE,),
    in_specs=[pl.BlockSpec((_BLK, 32), lambda i: (i, 0)),
              pl.BlockSpec((_BLK, 32), lambda i: (i, 0)),
              pl.BlockSpec((_BLK, 32), lambda i: (i, 0)),
              _cspec(32, 32), _cspec(1, 32)],
    out_specs=[pl.BlockSpec((_BLK, 32), lambda i: (i, 0)), _cspec(8, 32)],
    out_shape=[jax.ShapeDtypeStruct((N_EDGES, 32), jnp.float32),
               jax.ShapeDtypeStruct((8, 32), jnp.float32)],
)


def _mk_lin32(relu):
    def body(xin, w, b, yo, st):
        y = _dot(xin[...], w[...]) + b[...]
        if relu:
            y = jnp.maximum(y, 0.0)
        yo[...] = y
        _acc_stats(st, y)

    return pl.pallas_call(
        body,
        grid=(_GE,),
        in_specs=[pl.BlockSpec((_BLK, 32), lambda i: (i, 0)),
                  _cspec(32, 32), _cspec(1, 32)],
        out_specs=[pl.BlockSpec((_BLK, 32), lambda i: (i, 0)), _cspec(8, 32)],
        out_shape=[jax.ShapeDtypeStruct((N_EDGES, 32), jnp.float32),
                   jax.ShapeDtypeStruct((8, 32), jnp.float32)],
    )


_pe4_call = _mk_lin32(True)
_pe5_call = _mk_lin32(False)


def _pe6_body(g2r, h1r, a4, a1, cc, e1o):
    e1o[...] = g2r[...] * a4[...] + h1r[...] * a1[...] + cc[...]


_pe6_call = pl.pallas_call(
    _pe6_body,
    grid=(_G6,),
    in_specs=[pl.BlockSpec((_BLK6, 128), lambda i: (i, 0)),
              pl.BlockSpec((_BLK6, 128), lambda i: (i, 0)),
              _cspec(1, 128), _cspec(1, 128), _cspec(1, 128)],
    out_specs=pl.BlockSpec((_BLK6, 128), lambda i: (i, 0)),
    out_shape=jax.ShapeDtypeStruct((N_EDGES * 32 // 128, 128), jnp.float32),
)


def _pu_body(stt, w0, b0, g0, be0, w1, b1, g1, be1, dT, uo, ro):
    h = jnp.maximum(_dot(stt[...], w0[...]) + b0[...], 0.0)
    h = _bn_full(h, g0[...], be0[...])
    h = jnp.maximum(_dot(h, w1[...]) + b1[...], 0.0)
    u = _bn_full(h, g1[...], be1[...])
    uo[...] = u
    ro[...] = _dot(u, dT[...])


_pu_call = pl.pallas_call(
    _pu_body,
    out_shape=[jax.ShapeDtypeStruct((N_GRAPH, 32), jnp.float32),
               jax.ShapeDtypeStruct((N_GRAPH, 32), jnp.float32)],
)


def _pv_body(xx, bf, rtab, w0, b0, g0, be0, w1, b1, g1, be1, aT, bmT,
             vo, po, qo):
    h = jnp.maximum(_dot(xx[...], w0[...]) + b0[...], 0.0)
    h = _bn_full(h, g0[...], be0[...])
    h = jnp.maximum(_dot(h, w1[...]) + b1[...], 0.0)
    v = _bn_full(h, g1[...], be1[...])
    vo[...] = v
    oh = (bf[...] == lax.broadcasted_iota(
        jnp.int32, (N_GRAPH, N_NODES), 0).astype(jnp.float32)
          ).astype(jnp.float32)
    rn = lax.dot_general(oh, rtab[...], (((0,), (0,)), ((), ())),
                         preferred_element_type=jnp.float32)
    po[...] = _dot(v, aT[...]) + rn
    qo[...] = _dot(v, bmT[...])


_pv_call = pl.pallas_call(
    _pv_body,
    out_shape=[jax.ShapeDtypeStruct((N_NODES, 32), jnp.float32),
               jax.ShapeDtypeStruct((N_NODES, 32), jnp.float32),
               jax.ShapeDtypeStruct((N_NODES, 32), jnp.float32)],
)


def _pn_body2(v_, accp, cntp, u_, bf,
              wa, wb, wc, b0, g0, be0, w1, b1, g1, be1, w2, b2, g2, be2,
              sa, sb, sc_, c0, f0, fb0, c1, cb1, f1, fb1, c2, cb2, f2, fb2,
              v1o, u1o):
    acc = accp[0, :N_NODES, :] + accp[1, :N_NODES, :]
    cnt = cntp[0, :N_NODES, :] + cntp[1, :N_NODES, :]
    v = v_[...]
    u = u_[...]
    v_mean = acc / jnp.maximum(cnt, 1.0)
    oh = (bf[...] == lax.broadcasted_iota(
        jnp.int32, (N_GRAPH, N_NODES), 0).astype(jnp.float32)
          ).astype(jnp.float32)
    u_bn = lax.dot_general(oh, u, (((0,), (0,)), ((), ())),
                           preferred_element_type=jnp.float32)
    m = jnp.maximum(
        _dot(v, wa[...]) + _dot(v_mean, wb[...]) + _dot(u_bn, wc[...])
        + b0[...], 0.0)
    m = _bn_full(m, g0[...], be0[...])
    m = jnp.maximum(_dot(m, w1[...]) + b1[...], 0.0)
    m = _bn_full(m, g1[...], be1[...])
    m = _dot(m, w2[...]) + b2[...]
    v1 = _bn_full(m, g2[...], be2[...]) + v
    v1o[...] = v1
    u_e = _dot(oh, acc) / jnp.maximum(_dot(oh, cnt), 1.0)
    cntb = jnp.sum(oh, axis=1, keepdims=True)
    u_v = _dot(oh, v1) / jnp.maximum(cntb, 1.0)
    m = jnp.maximum(
        _dot(u_e, sa[...]) + _dot(u_v, sb[...]) + _dot(u, sc_[...])
        + c0[...], 0.0)
    m = _bn_full(m, f0[...], fb0[...])
    m = jnp.maximum(_dot(m, c1[...]) + cb1[...], 0.0)
    m = _bn_full(m, f1[...], fb1[...])
    m = _dot(m, c2[...]) + cb2[...]
    u1o[...] = _bn_full(m, f2[...], fb2[...]) + u


_pn_call = pl.pallas_call(
    _pn_body2,
    out_shape=[jax.ShapeDtypeStruct((N_NODES, 32), jnp.float32),
               jax.ShapeDtypeStruct((N_GRAPH, 32), jnp.float32)],
)


_SC_CACHE = {}


def _sc_kernels():
    if _SC_CACHE:
        return _SC_CACHE['gather'], _SC_CACHE['scatter']

    mesh = plsc.VectorSubcoreMesh(core_axis_name="c", subcore_axis_name="s",
                                  num_cores=NC, num_subcores=NS)

    @functools.partial(
        pl.kernel,
        mesh=mesh,
        out_type=[jax.ShapeDtypeStruct((N_EDGES, 32), jnp.float32),
                  jax.ShapeDtypeStruct((N_EDGES, 32), jnp.float32)],
        compiler_params=pltpu.CompilerParams(use_tc_tiling_on_sc=False),
        scratch_types=[pltpu.VMEM((SCK,), jnp.int32),
                       pltpu.VMEM((SCK, 32), jnp.float32),
                       pltpu.SemaphoreType.DMA],
    )
    def sc_gather(pp_hbm, q_hbm, src_hbm, dst_hbm, ts_hbm, td_hbm,
                  idx_v, rows_v, sem):
        c = lax.axis_index("c")
        s = lax.axis_index("s")
        base = (c * NS + s) * EPW

        def one_side(off, ihbm, thbm, ohbm):
            pltpu.sync_copy(ihbm.at[pl.ds(off, SCK)], idx_v)
            cps = [pltpu.async_copy(
                       thbm.at[idx_v.at[pl.ds(j * CH, CH)]],
                       rows_v.at[pl.ds(j * CH, CH)], sem)
                   for j in range(KSUB)]
            for cp in cps:
                cp.wait()
            pltpu.sync_copy(rows_v, ohbm.at[pl.ds(off, SCK)])

        def body(k, carry):
            off = base + k * SCK
            one_side(off, src_hbm, pp_hbm, ts_hbm)
            one_side(off, dst_hbm, q_hbm, td_hbm)
            return carry

        lax.fori_loop(0, NSCK, body, 0)

    @functools.partial(
        pl.kernel,
        mesh=mesh,
        out_type=[jax.ShapeDtypeStruct((NC, NPAD, 32), jnp.float32),
                  jax.ShapeDtypeStruct((NC, NPAD, 32), jnp.float32)],
        compiler_params=pltpu.CompilerParams(use_tc_tiling_on_sc=False),
        scratch_types=[pltpu.VMEM((KSUB, CH), jnp.int32),
                       pltpu.VMEM((SCK, 32), jnp.float32),
                       pltpu.VMEM((CH, 32), jnp.float32),
                       pltpu.VMEM((SL, 32), jnp.float32),
                       pltpu.VMEM_SHARED((NPAD, 32), jnp.float32),
                       pltpu.VMEM_SHARED((NPAD, 32), jnp.float32),
                       pltpu.SemaphoreType.DMA],
    )
    def sc_scatter(e1_hbm, src2d_hbm, zeros_hbm, ones_hbm, acc_hbm, cnt_hbm,
                   idx2_v, rows_v, ones_v, stage_v, accS, cntS, sem):
        c = lax.axis_index("c")
        s = lax.axis_index("s")
        base = (c * NS + s) * EPW
        brow = (c * NS + s) * NCH
        pltpu.sync_copy(zeros_hbm, stage_v)
        pltpu.sync_copy(stage_v, accS.at[pl.ds(s * SL, SL)])
        pltpu.sync_copy(stage_v, cntS.at[pl.ds(s * SL, SL)])
        pltpu.sync_copy(ones_hbm, ones_v)
        plsc.subcore_barrier()

        def body(k, carry):
            off = base + k * SCK
            pltpu.sync_copy(src2d_hbm.at[pl.ds(brow + k * KSUB, KSUB)], idx2_v)
            pltpu.sync_copy(e1_hbm.at[pl.ds(off, SCK)], rows_v)
            cps = []
            for j in range(KSUB):
                cps.append(pltpu.async_copy(
                    rows_v.at[pl.ds(j * CH, CH)],
                    accS.at[idx2_v.at[j]], sem, add=True))
                cps.append(pltpu.async_copy(
                    ones_v, cntS.at[idx2_v.at[j]], sem, add=True))
            for cp in cps:
                cp.wait()
            return carry

        lax.fori_loop(0, NSCK, body, 0)

        plsc.subcore_barrier()
        pltpu.sync_copy(accS.at[pl.ds(s * SL, SL)],
                        acc_hbm.at[c, pl.ds(s * SL, SL)])
        pltpu.sync_copy(cntS.at[pl.ds(s * SL, SL)],
                        cnt_hbm.at[c, pl.ds(s * SL, SL)])

    _SC_CACHE['gather'] = sc_gather
    _SC_CACHE['scatter'] = sc_scatter
    return sc_gather, sc_scatter


def _fold(st, n, g, be):
    m = st[0] / n
    var = st[1] / n - m * m
    sc = g / jnp.sqrt(var + 1e-5)
    sh = be - m * sc
    return sc, sh


@jax.jit
def kernel(x, edge_index, edge_attr, state, batch, params):
    src = edge_index[0].astype(jnp.int32)
    dst = edge_index[1].astype(jnp.int32)
    batchf = batch.astype(jnp.float32)[None, :]
    pe, pv, pu = params['e'], params['v'], params['u']
    pue, pun, pus = params['ue'], params['un'], params['us']
    w0ue = pue['W0']
    A, Bm, C, D = (w0ue[:, 0:32], w0ue[:, 32:64],
                   w0ue[:, 64:96], w0ue[:, 96:128])

    u, rtab = _pu_call(
        state, pu['W0'].T, pu['b0'][None], pu['g0'][None], pu['be0'][None],
        pu['W1'].T, pu['b1'][None], pu['g1'][None], pu['be1'][None], D.T)

    v, pp, q = _pv_call(
        x, batchf, rtab,
        pv['W0'].T, pv['b0'][None], pv['g0'][None], pv['be0'][None],
        pv['W1'].T, pv['b1'][None], pv['g1'][None], pv['be1'][None],
        A.T, Bm.T)

    nE = jnp.float32(N_EDGES)
    st0 = _pe1_call(edge_attr, pe['W0'].T, pe['b0'][None])
    s0, t0 = _fold(st0, nE, pe['g0'], pe['be0'])
    w1fT = (pe['W1'] * s0[None, :]).T
    b1f = (pe['b1'] + pe['W1'] @ t0)[None]
    h1, st1 = _pe2_call(edge_attr, pe['W0'].T, pe['b0'][None], w1fT, b1f)
    s1, t1 = _fold(st1, nE, pe['g1'], pe['be1'])

    sc_gather, sc_scatter = _sc_kernels()
    ts, td = sc_gather(pp, q, src, dst)

    c2T = (C * s1[None, :]).T
    c3 = (pue['b0'] + C @ t1)[None]
    g0, st2 = _pe3_call(h1, ts, td, c2T, c3)
    s2, t2 = _fold(st2, nE, pue['g0'], pue['be0'])

    w1f2T = (pue['W1'] * s2[None, :]).T
    b1f2 = (pue['b1'] + pue['W1'] @ t2)[None]
    g1, st3 = _pe4_call(g0, w1f2T, b1f2)
    s3, t3 = _fold(st3, nE, pue['g1'], pue['be1'])

    w2fT = (pue['W2'] * s3[None, :]).T
    b2f = (pue['b2'] + pue['W2'] @ t3)[None]
    g2, st4 = _pe5_call(g1, w2fT, b2f)
    s4, t4 = _fold(st4, nE, pue['g2'], pue['be2'])

    rr = N_EDGES * 32 // 128
    a4 = jnp.tile(s4, 4)[None]
    a1 = jnp.tile(s1, 4)[None]
    cc = jnp.tile(t4 + t1, 4)[None]
    e1r = _pe6_call(g2.reshape(rr, 128), h1.reshape(rr, 128), a4, a1, cc)
    e1 = e1r.reshape(N_EDGES, 32)

    zeros = jnp.zeros((SL, 32), jnp.float32)
    ones = jnp.ones((CH, 32), jnp.float32)
    src2d = src.reshape(N_EDGES // CH, CH)
    accp, cntp = sc_scatter(e1, src2d, zeros, ones)

    v1, u1 = _pn_call(
        v, accp, cntp, u, batchf,
        pun['W0'][:, 0:32].T, pun['W0'][:, 32:64].T, pun['W0'][:, 64:96].T,
        pun['b0'][None], pun['g0'][None], pun['be0'][None],
        pun['W1'].T, pun['b1'][None], pun['g1'][None], pun['be1'][None],
        pun['W2'].T, pun['b2'][None], pun['g2'][None], pun['be2'][None],
        pus['W0'][:, 0:32].T, pus['W0'][:, 32:64].T, pus['W0'][:, 64:96].T,
        pus['b0'][None], pus['g0'][None], pus['be0'][None],
        pus['W1'].T, pus['b1'][None], pus['g1'][None], pus['be1'][None],
        pus['W2'].T, pus['b2'][None], pus['g2'][None], pus['be2'][None])

    return (v1, e1, u1)


# speedup vs baseline: 4.4155x; 1.0244x over previous
import functools

import jax
import jax.numpy as jnp
from jax import lax
from jax.experimental import pallas as pl
from jax.experimental.pallas import tpu as pltpu
from jax.experimental.pallas import tpu_sc as plsc

N_NODES = 10000
N_EDGES = 320000
N_GRAPH = 64
NPAD = 10240
NC, NS = 2, 16
NW = NC * NS
EPW = N_EDGES // NW
CH = 80
NCH = EPW // CH
SL = NPAD // NS
SCK = 2000
KSUB = SCK // CH
NSCK = EPW // SCK

_BLK = 4000
_GE = N_EDGES // _BLK
_BLK6 = 8000
_G6 = (N_EDGES * 32) // 128 // _BLK6


def _acc_stats(st_ref, h):
    s = jnp.sum(h, axis=0, keepdims=True)
    ss = jnp.sum(h * h, axis=0, keepdims=True)
    part = jnp.concatenate(
        [s, ss, jnp.zeros((6, h.shape[1]), jnp.float32)], axis=0)
    i = pl.program_id(0)

    @pl.when(i == 0)
    def _():
        st_ref[...] = part

    @pl.when(i > 0)
    def _():
        st_ref[...] = st_ref[...] + part


def _dot(a, b):
    return jnp.dot(a, b, preferred_element_type=jnp.float32)


def _bn_full(h, g, be):
    m = jnp.mean(h, axis=0, keepdims=True)
    var = jnp.mean(h * h, axis=0, keepdims=True) - m * m
    sc = g / jnp.sqrt(var + 1e-5)
    return h * sc + (be - m * sc)


def _cspec(r, c):
    return pl.BlockSpec((r, c), lambda i: (0, 0))


def _pe1_body(ea, w0, b0, st):
    h0 = jnp.maximum(_dot(ea[...], w0[...]) + b0[...], 0.0)
    _acc_stats(st, h0)


_pe1_call = pl.pallas_call(
    _pe1_body,
    grid=(_GE,),
    in_specs=[pl.BlockSpec((_BLK, 16), lambda i: (i, 0)),
              _cspec(16, 64), _cspec(1, 64)],
    out_specs=_cspec(8, 64),
    out_shape=jax.ShapeDtypeStruct((8, 64), jnp.float32),
)


def _pe2_body(ea, w0, b0, w1, b1, h1o, st):
    h0 = jnp.maximum(_dot(ea[...], w0[...]) + b0[...], 0.0)
    h1 = jnp.maximum(_dot(h0, w1[...]) + b1[...], 0.0)
    h1o[...] = h1
    _acc_stats(st, h1)


_pe2_call = pl.pallas_call(
    _pe2_body,
    grid=(_GE,),
    in_specs=[pl.BlockSpec((_BLK, 16), lambda i: (i, 0)),
              _cspec(16, 64), _cspec(1, 64),
              _cspec(64, 32), _cspec(1, 32)],
    out_specs=[pl.BlockSpec((_BLK, 32), lambda i: (i, 0)), _cspec(8, 32)],
    out_shape=[jax.ShapeDtypeStruct((N_EDGES, 32), jnp.float32),
               jax.ShapeDtypeStruct((8, 32), jnp.float32)],
)


def _pe3_body(h1, ts, td, w, b, g0o, st):
    g0 = jnp.maximum(_dot(h1[...], w[...]) + ts[...] + td[...] + b[...], 0.0)
    g0o[...] = g0
    _acc_stats(st, g0)


_pe3_call = pl.pallas_call(
    _pe3_body,
    grid=(_GE,),
    in_specs=[pl.BlockSpec((_BLK, 32), lambda i: (i, 0)),
              pl.BlockSpec((_BLK, 32), lambda i: (i, 0)),
              pl.BlockSpec((_BLK, 32), lambda i: (i, 0)),
              _cspec(32, 32), _cspec(1, 32)],
    out_specs=[pl.BlockSpec((_BLK, 32), lambda i: (i, 0)), _cspec(8, 32)],
    out_shape=[jax.ShapeDtypeStruct((N_EDGES, 32), jnp.float32),
               jax.ShapeDtypeStruct((8, 32), jnp.float32)],
)


def _mk_lin32(relu):
    def body(xin, w, b, yo, st):
        y = _dot(xin[...], w[...]) + b[...]
        if relu:
            y = jnp.maximum(y, 0.0)
        yo[...] = y
        _acc_stats(st, y)

    return pl.pallas_call(
        body,
        grid=(_GE,),
        in_specs=[pl.BlockSpec((_BLK, 32), lambda i: (i, 0)),
                  _cspec(32, 32), _cspec(1, 32)],
        out_specs=[pl.BlockSpec((_BLK, 32), lambda i: (i, 0)), _cspec(8, 32)],
        out_shape=[jax.ShapeDtypeStruct((N_EDGES, 32), jnp.float32),
                   jax.ShapeDtypeStruct((8, 32), jnp.float32)],
    )


_pe4_call = _mk_lin32(True)
_pe5_call = _mk_lin32(False)


def _pe6_body(g2r, h1r, a4, a1, cc, e1o):
    e1o[...] = g2r[...] * a4[...] + h1r[...] * a1[...] + cc[...]


_pe6_call = pl.pallas_call(
    _pe6_body,
    grid=(_GE,),
    in_specs=[pl.BlockSpec((_BLK, 32), lambda i: (i, 0)),
              pl.BlockSpec((_BLK, 32), lambda i: (i, 0)),
              _cspec(1, 32), _cspec(1, 32), _cspec(1, 32)],
    out_specs=pl.BlockSpec((_BLK, 32), lambda i: (i, 0)),
    out_shape=jax.ShapeDtypeStruct((N_EDGES, 32), jnp.float32),
)


def _pv_body(xx, bf, stt, uw0, ub0, ug0, ube0, uw1, ub1, ug1, ube1, dT,
             w0, b0, g0, be0, w1, b1, g1, be1, aT, bmT,
             vo, po, qo, uo):
    h = jnp.maximum(_dot(stt[...], uw0[...]) + ub0[...], 0.0)
    h = _bn_full(h, ug0[...], ube0[...])
    h = jnp.maximum(_dot(h, uw1[...]) + ub1[...], 0.0)
    u = _bn_full(h, ug1[...], ube1[...])
    uo[...] = u
    rtab = _dot(u, dT[...])
    h = jnp.maximum(_dot(xx[...], w0[...]) + b0[...], 0.0)
    h = _bn_full(h, g0[...], be0[...])
    h = jnp.maximum(_dot(h, w1[...]) + b1[...], 0.0)
    v = _bn_full(h, g1[...], be1[...])
    vo[...] = v
    oh = (bf[...] == lax.broadcasted_iota(
        jnp.int32, (N_GRAPH, N_NODES), 0).astype(jnp.float32)
          ).astype(jnp.float32)
    rn = lax.dot_general(oh, rtab, (((0,), (0,)), ((), ())),
                         preferred_element_type=jnp.float32)
    po[...] = _dot(v, aT[...]) + rn
    qo[...] = _dot(v, bmT[...])


_pv_call = pl.pallas_call(
    _pv_body,
    out_shape=[jax.ShapeDtypeStruct((N_NODES, 32), jnp.float32),
               jax.ShapeDtypeStruct((N_NODES, 32), jnp.float32),
               jax.ShapeDtypeStruct((N_NODES, 32), jnp.float32),
               jax.ShapeDtypeStruct((N_GRAPH, 32), jnp.float32)],
)


def _pn_body2(v_, accp, cntp, u_, bf,
              wa, wb, wc, b0, g0, be0, w1, b1, g1, be1, w2, b2, g2, be2,
              sa, sb, sc_, c0, f0, fb0, c1, cb1, f1, fb1, c2, cb2, f2, fb2,
              v1o, u1o):
    acc = accp[0, :N_NODES, :] + accp[1, :N_NODES, :]
    cnt = cntp[0, :N_NODES, :] + cntp[1, :N_NODES, :]
    v = v_[...]
    u = u_[...]
    v_mean = acc / jnp.maximum(cnt, 1.0)
    oh = (bf[...] == lax.broadcasted_iota(
        jnp.int32, (N_GRAPH, N_NODES), 0).astype(jnp.float32)
          ).astype(jnp.float32)
    u_bn = lax.dot_general(oh, u, (((0,), (0,)), ((), ())),
                           preferred_element_type=jnp.float32)
    m = jnp.maximum(
        _dot(v, wa[...]) + _dot(v_mean, wb[...]) + _dot(u_bn, wc[...])
        + b0[...], 0.0)
    m = _bn_full(m, g0[...], be0[...])
    m = jnp.maximum(_dot(m, w1[...]) + b1[...], 0.0)
    m = _bn_full(m, g1[...], be1[...])
    m = _dot(m, w2[...]) + b2[...]
    v1 = _bn_full(m, g2[...], be2[...]) + v
    v1o[...] = v1
    u_e = _dot(oh, acc) / jnp.maximum(_dot(oh, cnt), 1.0)
    cntb = jnp.sum(oh, axis=1, keepdims=True)
    u_v = _dot(oh, v1) / jnp.maximum(cntb, 1.0)
    m = jnp.maximum(
        _dot(u_e, sa[...]) + _dot(u_v, sb[...]) + _dot(u, sc_[...])
        + c0[...], 0.0)
    m = _bn_full(m, f0[...], fb0[...])
    m = jnp.maximum(_dot(m, c1[...]) + cb1[...], 0.0)
    m = _bn_full(m, f1[...], fb1[...])
    m = _dot(m, c2[...]) + cb2[...]
    u1o[...] = _bn_full(m, f2[...], fb2[...]) + u


_pn_call = pl.pallas_call(
    _pn_body2,
    out_shape=[jax.ShapeDtypeStruct((N_NODES, 32), jnp.float32),
               jax.ShapeDtypeStruct((N_GRAPH, 32), jnp.float32)],
)


_SC_CACHE = {}


def _sc_kernels():
    if _SC_CACHE:
        return _SC_CACHE['gather'], _SC_CACHE['scatter']

    mesh = plsc.VectorSubcoreMesh(core_axis_name="c", subcore_axis_name="s",
                                  num_cores=NC, num_subcores=NS)

    @functools.partial(
        pl.kernel,
        mesh=mesh,
        out_type=[jax.ShapeDtypeStruct((N_EDGES, 32), jnp.float32),
                  jax.ShapeDtypeStruct((N_EDGES, 32), jnp.float32)],
        compiler_params=pltpu.CompilerParams(use_tc_tiling_on_sc=False),
        scratch_types=[pltpu.VMEM((KSUB, CH), jnp.int32),
                       pltpu.VMEM((SCK, 32), jnp.float32),
                       pltpu.SemaphoreType.DMA],
    )
    def sc_gather(pp_hbm, q_hbm, ei_hbm, ts_hbm, td_hbm,
                  idx2_v, rows_v, sem):
        c = lax.axis_index("c")
        s = lax.axis_index("s")
        base = (c * NS + s) * EPW
        brow = (c * NS + s) * NCH

        def one_side(off, row0, side, thbm, ohbm):
            pltpu.sync_copy(ei_hbm.at[side, pl.ds(row0, KSUB)], idx2_v)
            cps = [pltpu.async_copy(
                       thbm.at[idx2_v.at[j]],
                       rows_v.at[pl.ds(j * CH, CH)], sem)
                   for j in range(KSUB)]
            for cp in cps:
                cp.wait()
            pltpu.sync_copy(rows_v, ohbm.at[pl.ds(off, SCK)])

        def body(k, carry):
            off = base + k * SCK
            row0 = brow + k * KSUB
            one_side(off, row0, 0, pp_hbm, ts_hbm)
            one_side(off, row0, 1, q_hbm, td_hbm)
            return carry

        lax.fori_loop(0, NSCK, body, 0)

    @functools.partial(
        pl.kernel,
        mesh=mesh,
        out_type=[jax.ShapeDtypeStruct((NC, NPAD, 32), jnp.float32),
                  jax.ShapeDtypeStruct((NC, NPAD, 32), jnp.float32)],
        compiler_params=pltpu.CompilerParams(use_tc_tiling_on_sc=False),
        scratch_types=[pltpu.VMEM((KSUB, CH), jnp.int32),
                       pltpu.VMEM((SCK, 32), jnp.float32),
                       pltpu.VMEM((CH, 32), jnp.float32),
                       pltpu.VMEM((SL, 32), jnp.float32),
                       pltpu.VMEM_SHARED((NPAD, 32), jnp.float32),
                       pltpu.VMEM_SHARED((NPAD, 32), jnp.float32),
                       pltpu.SemaphoreType.DMA],
    )
    def sc_scatter(e1_hbm, ei_hbm, zeros_hbm, ones_hbm, acc_hbm, cnt_hbm,
                   idx2_v, rows_v, ones_v, stage_v, accS, cntS, sem):
        c = lax.axis_index("c")
        s = lax.axis_index("s")
        base = (c * NS + s) * EPW
        brow = (c * NS + s) * NCH
        pltpu.sync_copy(zeros_hbm, stage_v)
        pltpu.sync_copy(stage_v, accS.at[pl.ds(s * SL, SL)])
        pltpu.sync_copy(stage_v, cntS.at[pl.ds(s * SL, SL)])
        pltpu.sync_copy(ones_hbm, ones_v)
        plsc.subcore_barrier()

        def body(k, carry):
            off = base + k * SCK
            pltpu.sync_copy(ei_hbm.at[0, pl.ds(brow + k * KSUB, KSUB)], idx2_v)
            pltpu.sync_copy(e1_hbm.at[pl.ds(off, SCK)], rows_v)
            cps = []
            for j in range(KSUB):
                cps.append(pltpu.async_copy(
                    rows_v.at[pl.ds(j * CH, CH)],
                    accS.at[idx2_v.at[j]], sem, add=True))
                cps.append(pltpu.async_copy(
                    ones_v, cntS.at[idx2_v.at[j]], sem, add=True))
            for cp in cps:
                cp.wait()
            return carry

        lax.fori_loop(0, NSCK, body, 0)

        plsc.subcore_barrier()
        pltpu.sync_copy(accS.at[pl.ds(s * SL, SL)],
                        acc_hbm.at[c, pl.ds(s * SL, SL)])
        pltpu.sync_copy(cntS.at[pl.ds(s * SL, SL)],
                        cnt_hbm.at[c, pl.ds(s * SL, SL)])

    _SC_CACHE['gather'] = sc_gather
    _SC_CACHE['scatter'] = sc_scatter
    return sc_gather, sc_scatter


def _fold(st, n, g, be):
    m = st[0] / n
    var = st[1] / n - m * m
    sc = g / jnp.sqrt(var + 1e-5)
    sh = be - m * sc
    return sc, sh


@jax.jit
def kernel(x, edge_index, edge_attr, state, batch, params):
    ei3 = edge_index.astype(jnp.int32).reshape(2, N_EDGES // CH, CH)
    batchf = batch.astype(jnp.float32)[None, :]
    pe, pv, pu = params['e'], params['v'], params['u']
    pue, pun, pus = params['ue'], params['un'], params['us']
    w0ue = pue['W0']
    A, Bm, C, D = (w0ue[:, 0:32], w0ue[:, 32:64],
                   w0ue[:, 64:96], w0ue[:, 96:128])

    v, pp, q, u = _pv_call(
        x, batchf, state,
        pu['W0'].T, pu['b0'][None], pu['g0'][None], pu['be0'][None],
        pu['W1'].T, pu['b1'][None], pu['g1'][None], pu['be1'][None], D.T,
        pv['W0'].T, pv['b0'][None], pv['g0'][None], pv['be0'][None],
        pv['W1'].T, pv['b1'][None], pv['g1'][None], pv['be1'][None],
        A.T, Bm.T)

    nE = jnp.float32(N_EDGES)
    st0 = _pe1_call(edge_attr, pe['W0'].T, pe['b0'][None])
    s0, t0 = _fold(st0, nE, pe['g0'], pe['be0'])
    w1fT = (pe['W1'] * s0[None, :]).T
    b1f = (pe['b1'] + pe['W1'] @ t0)[None]
    h1, st1 = _pe2_call(edge_attr, pe['W0'].T, pe['b0'][None], w1fT, b1f)
    s1, t1 = _fold(st1, nE, pe['g1'], pe['be1'])

    sc_gather, sc_scatter = _sc_kernels()
    ts, td = sc_gather(pp, q, ei3)

    c2T = (C * s1[None, :]).T
    c3 = (pue['b0'] + C @ t1)[None]
    g0, st2 = _pe3_call(h1, ts, td, c2T, c3)
    s2, t2 = _fold(st2, nE, pue['g0'], pue['be0'])

    w1f2T = (pue['W1'] * s2[None, :]).T
    b1f2 = (pue['b1'] + pue['W1'] @ t2)[None]
    g1, st3 = _pe4_call(g0, w1f2T, b1f2)
    s3, t3 = _fold(st3, nE, pue['g1'], pue['be1'])

    w2fT = (pue['W2'] * s3[None, :]).T
    b2f = (pue['b2'] + pue['W2'] @ t3)[None]
    g2, st4 = _pe5_call(g1, w2fT, b2f)
    s4, t4 = _fold(st4, nE, pue['g2'], pue['be2'])

    e1 = _pe6_call(g2, h1, s4[None], s1[None], (t4 + t1)[None])

    zeros = jnp.zeros((SL, 32), jnp.float32)
    ones = jnp.ones((CH, 32), jnp.float32)
    accp, cntp = sc_scatter(e1, ei3, zeros, ones)

    v1, u1 = _pn_call(
        v, accp, cntp, u, batchf,
        pun['W0'][:, 0:32].T, pun['W0'][:, 32:64].T, pun['W0'][:, 64:96].T,
        pun['b0'][None], pun['g0'][None], pun['be0'][None],
        pun['W1'].T, pun['b1'][None], pun['g1'][None], pun['be1'][None],
        pun['W2'].T, pun['b2'][None], pun['g2'][None], pun['be2'][None],
        pus['W0'][:, 0:32].T, pus['W0'][:, 32:64].T, pus['W0'][:, 64:96].T,
        pus['b0'][None], pus['g0'][None], pus['be0'][None],
        pus['W1'].T, pus['b1'][None], pus['g1'][None], pus['be1'][None],
        pus['W2'].T, pus['b2'][None], pus['g2'][None], pus['be2'][None])

    return (v1, e1, u1)


# speedup vs baseline: 8.5968x; 1.9470x over previous
import functools

import jax
import jax.numpy as jnp
from jax import lax
from jax.experimental import pallas as pl
from jax.experimental.pallas import tpu as pltpu
from jax.experimental.pallas import tpu_sc as plsc

N_NODES = 10000
N_EDGES = 320000
N_GRAPH = 64
NPAD = 10240
NC, NS = 2, 16
NW = NC * NS
EPW = N_EDGES // NW
CH = 80
NCH = EPW // CH
SL = NPAD // NS
SCK = 2000
KSUB = SCK // CH
NSCK = EPW // SCK

_PK = 8
_EP = N_EDGES // _PK
_BLK = 2000
_GE = _EP // _BLK


def _acc_stats(st_ref, h):
    s = jnp.sum(h, axis=0, keepdims=True)
    ss = jnp.sum(h * h, axis=0, keepdims=True)
    part = jnp.concatenate(
        [s, ss, jnp.zeros((6, h.shape[1]), jnp.float32)], axis=0)
    i = pl.program_id(0)

    @pl.when(i == 0)
    def _():
        st_ref[...] = part

    @pl.when(i > 0)
    def _():
        st_ref[...] = st_ref[...] + part


def _dot(a, b):
    return jnp.dot(a, b, preferred_element_type=jnp.float32)


def _bn_full(h, g, be):
    m = jnp.mean(h, axis=0, keepdims=True)
    var = jnp.mean(h * h, axis=0, keepdims=True) - m * m
    sc = g / jnp.sqrt(var + 1e-5)
    return h * sc + (be - m * sc)


def _cspec(r, c):
    return pl.BlockSpec((r, c), lambda i: (0, 0))


def _bspec(c):
    return pl.BlockSpec((_BLK, c), lambda i: (i, 0))


def _pe1_body(ea, w0, b0, st):
    h0 = jnp.maximum(_dot(ea[...], w0[...]) + b0[...], 0.0)
    _acc_stats(st, h0)


_pe1_call = pl.pallas_call(
    _pe1_body,
    grid=(_GE,),
    in_specs=[_bspec(128), _cspec(128, 512), _cspec(1, 512)],
    out_specs=_cspec(8, 512),
    out_shape=jax.ShapeDtypeStruct((8, 512), jnp.float32),
)


def _pe2_body(ea, w0, b0, w1, b1, h1o, st):
    h0 = jnp.maximum(_dot(ea[...], w0[...]) + b0[...], 0.0)
    h1 = jnp.maximum(_dot(h0, w1[...]) + b1[...], 0.0)
    h1o[...] = h1
    _acc_stats(st, h1)


_pe2_call = pl.pallas_call(
    _pe2_body,
    grid=(_GE,),
    in_specs=[_bspec(128), _cspec(128, 512), _cspec(1, 512),
              _cspec(512, 256), _cspec(1, 256)],
    out_specs=[_bspec(256), _cspec(8, 256)],
    out_shape=[jax.ShapeDtypeStruct((_EP, 256), jnp.float32),
               jax.ShapeDtypeStruct((8, 256), jnp.float32)],
)


def _pe3_body(h1, ts, td, w, b, g0o, st):
    g0 = jnp.maximum(_dot(h1[...], w[...]) + ts[...] + td[...] + b[...], 0.0)
    g0o[...] = g0
    _acc_stats(st, g0)


_pe3_call = pl.pallas_call(
    _pe3_body,
    grid=(_GE,),
    in_specs=[_bspec(256), _bspec(256), _bspec(256),
              _cspec(256, 256), _cspec(1, 256)],
    out_specs=[_bspec(256), _cspec(8, 256)],
    out_shape=[jax.ShapeDtypeStruct((_EP, 256), jnp.float32),
               jax.ShapeDtypeStruct((8, 256), jnp.float32)],
)


def _mk_lin32(relu):
    def body(xin, w, b, yo, st):
        y = _dot(xin[...], w[...]) + b[...]
        if relu:
            y = jnp.maximum(y, 0.0)
        yo[...] = y
        _acc_stats(st, y)

    return pl.pallas_call(
        body,
        grid=(_GE,),
        in_specs=[_bspec(256), _cspec(256, 256), _cspec(1, 256)],
        out_specs=[_bspec(256), _cspec(8, 256)],
        out_shape=[jax.ShapeDtypeStruct((_EP, 256), jnp.float32),
                   jax.ShapeDtypeStruct((8, 256), jnp.float32)],
    )


_pe4_call = _mk_lin32(True)
_pe5_call = _mk_lin32(False)


def _pe6_body(g2r, h1r, a4, a1, cc, e1o):
    e1o[...] = g2r[...] * a4[...] + h1r[...] * a1[...] + cc[...]


_pe6_call = pl.pallas_call(
    _pe6_body,
    grid=(_GE,),
    in_specs=[_bspec(256), _bspec(256),
              _cspec(1, 256), _cspec(1, 256), _cspec(1, 256)],
    out_specs=_bspec(256),
    out_shape=jax.ShapeDtypeStruct((_EP, 256), jnp.float32),
)


def _pv_body(xx, bf, stt, uw0, ub0, ug0, ube0, uw1, ub1, ug1, ube1, dT,
             w0, b0, g0, be0, w1, b1, g1, be1, aT, bmT,
             vo, po, qo, uo):
    h = jnp.maximum(_dot(stt[...], uw0[...]) + ub0[...], 0.0)
    h = _bn_full(h, ug0[...], ube0[...])
    h = jnp.maximum(_dot(h, uw1[...]) + ub1[...], 0.0)
    u = _bn_full(h, ug1[...], ube1[...])
    uo[...] = u
    rtab = _dot(u, dT[...])
    h = jnp.maximum(_dot(xx[...], w0[...]) + b0[...], 0.0)
    h = _bn_full(h, g0[...], be0[...])
    h = jnp.maximum(_dot(h, w1[...]) + b1[...], 0.0)
    v = _bn_full(h, g1[...], be1[...])
    vo[...] = v
    oh = (bf[...] == lax.broadcasted_iota(
        jnp.int32, (N_GRAPH, N_NODES), 0).astype(jnp.float32)
          ).astype(jnp.float32)
    rn = lax.dot_general(oh, rtab, (((0,), (0,)), ((), ())),
                         preferred_element_type=jnp.float32)
    po[...] = _dot(v, aT[...]) + rn
    qo[...] = _dot(v, bmT[...])


_pv_call = pl.pallas_call(
    _pv_body,
    out_shape=[jax.ShapeDtypeStruct((N_NODES, 32), jnp.float32),
               jax.ShapeDtypeStruct((N_NODES, 32), jnp.float32),
               jax.ShapeDtypeStruct((N_NODES, 32), jnp.float32),
               jax.ShapeDtypeStruct((N_GRAPH, 32), jnp.float32)],
)


def _pn_body2(v_, accp, cntp, u_, bf,
              wa, wb, wc, b0, g0, be0, w1, b1, g1, be1, w2, b2, g2, be2,
              sa, sb, sc_, c0, f0, fb0, c1, cb1, f1, fb1, c2, cb2, f2, fb2,
              v1o, u1o):
    acc = accp[0, :N_NODES, :] + accp[1, :N_NODES, :]
    cnt = cntp[0, :N_NODES, :] + cntp[1, :N_NODES, :]
    v = v_[...]
    u = u_[...]
    v_mean = acc / jnp.maximum(cnt, 1.0)
    oh = (bf[...] == lax.broadcasted_iota(
        jnp.int32, (N_GRAPH, N_NODES), 0).astype(jnp.float32)
          ).astype(jnp.float32)
    u_bn = lax.dot_general(oh, u, (((0,), (0,)), ((), ())),
                           preferred_element_type=jnp.float32)
    m = jnp.maximum(
        _dot(v, wa[...]) + _dot(v_mean, wb[...]) + _dot(u_bn, wc[...])
        + b0[...], 0.0)
    m = _bn_full(m, g0[...], be0[...])
    m = jnp.maximum(_dot(m, w1[...]) + b1[...], 0.0)
    m = _bn_full(m, g1[...], be1[...])
    m = _dot(m, w2[...]) + b2[...]
    v1 = _bn_full(m, g2[...], be2[...]) + v
    v1o[...] = v1
    u_e = _dot(oh, acc) / jnp.maximum(_dot(oh, cnt), 1.0)
    cntb = jnp.sum(oh, axis=1, keepdims=True)
    u_v = _dot(oh, v1) / jnp.maximum(cntb, 1.0)
    m = jnp.maximum(
        _dot(u_e, sa[...]) + _dot(u_v, sb[...]) + _dot(u, sc_[...])
        + c0[...], 0.0)
    m = _bn_full(m, f0[...], fb0[...])
    m = jnp.maximum(_dot(m, c1[...]) + cb1[...], 0.0)
    m = _bn_full(m, f1[...], fb1[...])
    m = _dot(m, c2[...]) + cb2[...]
    u1o[...] = _bn_full(m, f2[...], fb2[...]) + u


_pn_call = pl.pallas_call(
    _pn_body2,
    out_shape=[jax.ShapeDtypeStruct((N_NODES, 32), jnp.float32),
               jax.ShapeDtypeStruct((N_GRAPH, 32), jnp.float32)],
)


_SC_CACHE = {}


def _sc_kernels():
    if _SC_CACHE:
        return _SC_CACHE['gather'], _SC_CACHE['scatter']

    mesh = plsc.VectorSubcoreMesh(core_axis_name="c", subcore_axis_name="s",
                                  num_cores=NC, num_subcores=NS)

    @functools.partial(
        pl.kernel,
        mesh=mesh,
        out_type=[jax.ShapeDtypeStruct((N_EDGES, 32), jnp.float32),
                  jax.ShapeDtypeStruct((N_EDGES, 32), jnp.float32)],
        compiler_params=pltpu.CompilerParams(use_tc_tiling_on_sc=False),
        scratch_types=[pltpu.VMEM((KSUB, CH), jnp.int32),
                       pltpu.VMEM((SCK, 32), jnp.float32),
                       pltpu.SemaphoreType.DMA],
    )
    def sc_gather(pp_hbm, q_hbm, ei_hbm, ts_hbm, td_hbm,
                  idx2_v, rows_v, sem):
        c = lax.axis_index("c")
        s = lax.axis_index("s")
        base = (c * NS + s) * EPW
        brow = (c * NS + s) * NCH

        def one_side(off, row0, side, thbm, ohbm):
            pltpu.sync_copy(ei_hbm.at[side, pl.ds(row0, KSUB)], idx2_v)
            cps = [pltpu.async_copy(
                       thbm.at[idx2_v.at[j]],
                       rows_v.at[pl.ds(j * CH, CH)], sem)
                   for j in range(KSUB)]
            for cp in cps:
                cp.wait()
            pltpu.sync_copy(rows_v, ohbm.at[pl.ds(off, SCK)])

        def body(k, carry):
            off = base + k * SCK
            row0 = brow + k * KSUB
            one_side(off, row0, 0, pp_hbm, ts_hbm)
            one_side(off, row0, 1, q_hbm, td_hbm)
            return carry

        lax.fori_loop(0, NSCK, body, 0)

    @functools.partial(
        pl.kernel,
        mesh=mesh,
        out_type=[jax.ShapeDtypeStruct((NC, NPAD, 32), jnp.float32),
                  jax.ShapeDtypeStruct((NC, NPAD, 32), jnp.float32)],
        compiler_params=pltpu.CompilerParams(use_tc_tiling_on_sc=False),
        scratch_types=[pltpu.VMEM((KSUB, CH), jnp.int32),
                       pltpu.VMEM((SCK, 32), jnp.float32),
                       pltpu.VMEM((CH, 32), jnp.float32),
                       pltpu.VMEM((SL, 32), jnp.float32),
                       pltpu.VMEM_SHARED((NPAD, 32), jnp.float32),
                       pltpu.VMEM_SHARED((NPAD, 32), jnp.float32),
                       pltpu.SemaphoreType.DMA],
    )
    def sc_scatter(e1_hbm, ei_hbm, zeros_hbm, ones_hbm, acc_hbm, cnt_hbm,
                   idx2_v, rows_v, ones_v, stage_v, accS, cntS, sem):
        c = lax.axis_index("c")
        s = lax.axis_index("s")
        base = (c * NS + s) * EPW
        brow = (c * NS + s) * NCH
        pltpu.sync_copy(zeros_hbm, stage_v)
        pltpu.sync_copy(stage_v, accS.at[pl.ds(s * SL, SL)])
        pltpu.sync_copy(stage_v, cntS.at[pl.ds(s * SL, SL)])
        pltpu.sync_copy(ones_hbm, ones_v)
        plsc.subcore_barrier()

        def body(k, carry):
            off = base + k * SCK
            pltpu.sync_copy(ei_hbm.at[0, pl.ds(brow + k * KSUB, KSUB)], idx2_v)
            pltpu.sync_copy(e1_hbm.at[pl.ds(off, SCK)], rows_v)
            cps = []
            for j in range(KSUB):
                cps.append(pltpu.async_copy(
                    rows_v.at[pl.ds(j * CH, CH)],
                    accS.at[idx2_v.at[j]], sem, add=True))
                cps.append(pltpu.async_copy(
                    ones_v, cntS.at[idx2_v.at[j]], sem, add=True))
            for cp in cps:
                cp.wait()
            return carry

        lax.fori_loop(0, NSCK, body, 0)

        plsc.subcore_barrier()
        pltpu.sync_copy(accS.at[pl.ds(s * SL, SL)],
                        acc_hbm.at[c, pl.ds(s * SL, SL)])
        pltpu.sync_copy(cntS.at[pl.ds(s * SL, SL)],
                        cnt_hbm.at[c, pl.ds(s * SL, SL)])

    _SC_CACHE['gather'] = sc_gather
    _SC_CACHE['scatter'] = sc_scatter
    return sc_gather, sc_scatter


def _fold(st, n, g, be):
    m = st[0] / n
    var = st[1] / n - m * m
    sc = g / jnp.sqrt(var + 1e-5)
    sh = be - m * sc
    return sc, sh


@jax.jit
def kernel(x, edge_index, edge_attr, state, batch, params):
    ei3 = edge_index.astype(jnp.int32).reshape(2, N_EDGES // CH, CH)
    batchf = batch.astype(jnp.float32)[None, :]
    pe, pv, pu = params['e'], params['v'], params['u']
    pue, pun, pus = params['ue'], params['un'], params['us']
    w0ue = pue['W0']
    A, Bm, C, D = (w0ue[:, 0:32], w0ue[:, 32:64],
                   w0ue[:, 64:96], w0ue[:, 96:128])

    v, pp, q, u = _pv_call(
        x, batchf, state,
        pu['W0'].T, pu['b0'][None], pu['g0'][None], pu['be0'][None],
        pu['W1'].T, pu['b1'][None], pu['g1'][None], pu['be1'][None], D.T,
        pv['W0'].T, pv['b0'][None], pv['g0'][None], pv['be0'][None],
        pv['W1'].T, pv['b1'][None], pv['g1'][None], pv['be1'][None],
        A.T, Bm.T)

    nE = jnp.float32(N_EDGES)
    eye = jnp.eye(_PK, dtype=jnp.float32)

    def bd(w):
        return jnp.kron(eye, w)

    def t8(b):
        return jnp.tile(b, _PK)[None]

    def fold8(st, d, g, be):
        m = st[0].reshape(_PK, d).sum(0) / nE
        var = st[1].reshape(_PK, d).sum(0) / nE - m * m
        sc = g / jnp.sqrt(var + 1e-5)
        return sc, be - m * sc

    ea_p = edge_attr.reshape(_EP, _PK * 16)
    w0bd = bd(pe['W0'].T)
    b0t = t8(pe['b0'])
    st0 = _pe1_call(ea_p, w0bd, b0t)
    s0, t0 = fold8(st0, 64, pe['g0'], pe['be0'])
    w1fT = (pe['W1'] * s0[None, :]).T
    b1f = pe['b1'] + pe['W1'] @ t0
    h1p, st1 = _pe2_call(ea_p, w0bd, b0t, bd(w1fT), t8(b1f))
    s1, t1 = fold8(st1, 32, pe['g1'], pe['be1'])

    sc_gather, sc_scatter = _sc_kernels()
    ts, td = sc_gather(pp, q, ei3)

    c2T = (C * s1[None, :]).T
    c3 = pue['b0'] + C @ t1
    g0p, st2 = _pe3_call(h1p, ts.reshape(_EP, 256), td.reshape(_EP, 256),
                         bd(c2T), t8(c3))
    s2, t2 = fold8(st2, 32, pue['g0'], pue['be0'])

    w1f2T = (pue['W1'] * s2[None, :]).T
    b1f2 = pue['b1'] + pue['W1'] @ t2
    g1p, st3 = _pe4_call(g0p, bd(w1f2T), t8(b1f2))
    s3, t3 = fold8(st3, 32, pue['g1'], pue['be1'])

    w2fT = (pue['W2'] * s3[None, :]).T
    b2f = pue['b2'] + pue['W2'] @ t3
    g2p, st4 = _pe5_call(g1p, bd(w2fT), t8(b2f))
    s4, t4 = fold8(st4, 32, pue['g2'], pue['be2'])

    e1p = _pe6_call(g2p, h1p, t8(s4), t8(s1), t8(t4 + t1))
    e1 = e1p.reshape(N_EDGES, 32)

    zeros = jnp.zeros((SL, 32), jnp.float32)
    ones = jnp.ones((CH, 32), jnp.float32)
    accp, cntp = sc_scatter(e1, ei3, zeros, ones)

    v1, u1 = _pn_call(
        v, accp, cntp, u, batchf,
        pun['W0'][:, 0:32].T, pun['W0'][:, 32:64].T, pun['W0'][:, 64:96].T,
        pun['b0'][None], pun['g0'][None], pun['be0'][None],
        pun['W1'].T, pun['b1'][None], pun['g1'][None], pun['be1'][None],
        pun['W2'].T, pun['b2'][None], pun['g2'][None], pun['be2'][None],
        pus['W0'][:, 0:32].T, pus['W0'][:, 32:64].T, pus['W0'][:, 64:96].T,
        pus['b0'][None], pus['g0'][None], pus['be0'][None],
        pus['W1'].T, pus['b1'][None], pus['g1'][None], pus['be1'][None],
        pus['W2'].T, pus['b2'][None], pus['g2'][None], pus['be2'][None])

    return (v1, e1, u1)


# speedup vs baseline: 8.6092x; 1.0014x over previous
import functools

import jax
import jax.numpy as jnp
from jax import lax
from jax.experimental import pallas as pl
from jax.experimental.pallas import tpu as pltpu
from jax.experimental.pallas import tpu_sc as plsc

N_NODES = 10000
N_EDGES = 320000
N_GRAPH = 64
NPAD = 10240
NC, NS = 2, 16
NW = NC * NS
EPW = N_EDGES // NW
CH = 80
NCH = EPW // CH
SL = NPAD // NS
SCK = 2000
KSUB = SCK // CH
NSCK = EPW // SCK

_PK = 8
_EP = N_EDGES // _PK
_BLK = 2000
_GE = _EP // _BLK


def _acc_stats(st_ref, h):
    s = jnp.sum(h, axis=0, keepdims=True)
    ss = jnp.sum(h * h, axis=0, keepdims=True)
    part = jnp.concatenate(
        [s, ss, jnp.zeros((6, h.shape[1]), jnp.float32)], axis=0)
    i = pl.program_id(0)

    @pl.when(i == 0)
    def _():
        st_ref[...] = part

    @pl.when(i > 0)
    def _():
        st_ref[...] = st_ref[...] + part


def _dot(a, b):
    return jnp.dot(a, b, preferred_element_type=jnp.float32)


def _bn_full(h, g, be):
    m = jnp.mean(h, axis=0, keepdims=True)
    var = jnp.mean(h * h, axis=0, keepdims=True) - m * m
    sc = g / jnp.sqrt(var + 1e-5)
    return h * sc + (be - m * sc)


def _cspec(r, c):
    return pl.BlockSpec((r, c), lambda i: (0, 0))


def _bspec(c):
    return pl.BlockSpec((_BLK, c), lambda i: (i, 0))


def _pe1_body(ea, w0, b0, st):
    h0 = jnp.maximum(_dot(ea[...], w0[...]) + b0[...], 0.0)
    _acc_stats(st, h0)


_pe1_call = pl.pallas_call(
    _pe1_body,
    grid=(_GE,),
    in_specs=[_bspec(128), _cspec(128, 512), _cspec(1, 512)],
    out_specs=_cspec(8, 512),
    out_shape=jax.ShapeDtypeStruct((8, 512), jnp.float32),
)


def _pe2_body(ea, w0, b0, w1, b1, h1o, st):
    h0 = jnp.maximum(_dot(ea[...], w0[...]) + b0[...], 0.0)
    h1 = jnp.maximum(_dot(h0, w1[...]) + b1[...], 0.0)
    h1o[...] = h1
    _acc_stats(st, h1)


_pe2_call = pl.pallas_call(
    _pe2_body,
    grid=(_GE,),
    in_specs=[_bspec(128), _cspec(128, 512), _cspec(1, 512),
              _cspec(512, 256), _cspec(1, 256)],
    out_specs=[_bspec(256), _cspec(8, 256)],
    out_shape=[jax.ShapeDtypeStruct((_EP, 256), jnp.float32),
               jax.ShapeDtypeStruct((8, 256), jnp.float32)],
)


def _pe3_body(h1, ts, td, w, b, g0o, st):
    g0 = jnp.maximum(_dot(h1[...], w[...]) + ts[...] + td[...] + b[...], 0.0)
    g0o[...] = g0
    _acc_stats(st, g0)


_pe3_call = pl.pallas_call(
    _pe3_body,
    grid=(_GE,),
    in_specs=[_bspec(256), _bspec(256), _bspec(256),
              _cspec(256, 256), _cspec(1, 256)],
    out_specs=[_bspec(256), _cspec(8, 256)],
    out_shape=[jax.ShapeDtypeStruct((_EP, 256), jnp.float32),
               jax.ShapeDtypeStruct((8, 256), jnp.float32)],
)


def _mk_lin32(relu):
    def body(xin, w, b, yo, st):
        y = _dot(xin[...], w[...]) + b[...]
        if relu:
            y = jnp.maximum(y, 0.0)
        yo[...] = y
        _acc_stats(st, y)

    return pl.pallas_call(
        body,
        grid=(_GE,),
        in_specs=[_bspec(256), _cspec(256, 256), _cspec(1, 256)],
        out_specs=[_bspec(256), _cspec(8, 256)],
        out_shape=[jax.ShapeDtypeStruct((_EP, 256), jnp.float32),
                   jax.ShapeDtypeStruct((8, 256), jnp.float32)],
    )


_pe4_call = _mk_lin32(True)
_pe5_call = _mk_lin32(False)


def _pe6_body(g2r, h1r, a4, a1, cc, e1o):
    e1o[...] = g2r[...] * a4[...] + h1r[...] * a1[...] + cc[...]


_pe6_call = pl.pallas_call(
    _pe6_body,
    grid=(_GE,),
    in_specs=[_bspec(256), _bspec(256),
              _cspec(1, 256), _cspec(1, 256), _cspec(1, 256)],
    out_specs=_bspec(256),
    out_shape=jax.ShapeDtypeStruct((_EP, 256), jnp.float32),
)


def _pv_body(xx, bf, stt, uw0, ub0, ug0, ube0, uw1, ub1, ug1, ube1, dT,
             w0, b0, g0, be0, w1, b1, g1, be1, aT, bmT,
             vo, po, qo, uo):
    h = jnp.maximum(_dot(stt[...], uw0[...]) + ub0[...], 0.0)
    h = _bn_full(h, ug0[...], ube0[...])
    h = jnp.maximum(_dot(h, uw1[...]) + ub1[...], 0.0)
    u = _bn_full(h, ug1[...], ube1[...])
    uo[...] = u
    rtab = _dot(u, dT[...])
    h = jnp.maximum(_dot(xx[...], w0[...]) + b0[...], 0.0)
    h = _bn_full(h, g0[...], be0[...])
    h = jnp.maximum(_dot(h, w1[...]) + b1[...], 0.0)
    v = _bn_full(h, g1[...], be1[...])
    vo[...] = v
    oh = (bf[...] == lax.broadcasted_iota(
        jnp.int32, (N_GRAPH, N_NODES), 0).astype(jnp.float32)
          ).astype(jnp.float32)
    rn = lax.dot_general(oh, rtab, (((0,), (0,)), ((), ())),
                         preferred_element_type=jnp.float32)
    po[...] = _dot(v, aT[...]) + rn
    qo[...] = _dot(v, bmT[...])


_pv_call = pl.pallas_call(
    _pv_body,
    out_shape=[jax.ShapeDtypeStruct((N_NODES, 32), jnp.float32),
               jax.ShapeDtypeStruct((N_NODES, 32), jnp.float32),
               jax.ShapeDtypeStruct((N_NODES, 32), jnp.float32),
               jax.ShapeDtypeStruct((N_GRAPH, 32), jnp.float32)],
)


def _pn_body2(v_, accp, cntp, u_, bf,
              wa, wb, wc, b0, g0, be0, w1, b1, g1, be1, w2, b2, g2, be2,
              sa, sb, sc_, c0, f0, fb0, c1, cb1, f1, fb1, c2, cb2, f2, fb2,
              v1o, u1o):
    acc = accp[0, :N_NODES, :] + accp[1, :N_NODES, :]
    cnt = cntp[0, :N_NODES, :] + cntp[1, :N_NODES, :]
    v = v_[...]
    u = u_[...]
    v_mean = acc / jnp.maximum(cnt, 1.0)
    oh = (bf[...] == lax.broadcasted_iota(
        jnp.int32, (N_GRAPH, N_NODES), 0).astype(jnp.float32)
          ).astype(jnp.float32)
    u_bn = lax.dot_general(oh, u, (((0,), (0,)), ((), ())),
                           preferred_element_type=jnp.float32)
    m = jnp.maximum(
        _dot(v, wa[...]) + _dot(v_mean, wb[...]) + _dot(u_bn, wc[...])
        + b0[...], 0.0)
    m = _bn_full(m, g0[...], be0[...])
    m = jnp.maximum(_dot(m, w1[...]) + b1[...], 0.0)
    m = _bn_full(m, g1[...], be1[...])
    m = _dot(m, w2[...]) + b2[...]
    v1 = _bn_full(m, g2[...], be2[...]) + v
    v1o[...] = v1
    u_e = _dot(oh, acc) / jnp.maximum(_dot(oh, cnt), 1.0)
    cntb = jnp.sum(oh, axis=1, keepdims=True)
    u_v = _dot(oh, v1) / jnp.maximum(cntb, 1.0)
    m = jnp.maximum(
        _dot(u_e, sa[...]) + _dot(u_v, sb[...]) + _dot(u, sc_[...])
        + c0[...], 0.0)
    m = _bn_full(m, f0[...], fb0[...])
    m = jnp.maximum(_dot(m, c1[...]) + cb1[...], 0.0)
    m = _bn_full(m, f1[...], fb1[...])
    m = _dot(m, c2[...]) + cb2[...]
    u1o[...] = _bn_full(m, f2[...], fb2[...]) + u


_pn_call = pl.pallas_call(
    _pn_body2,
    out_shape=[jax.ShapeDtypeStruct((N_NODES, 32), jnp.float32),
               jax.ShapeDtypeStruct((N_GRAPH, 32), jnp.float32)],
)


_SC_CACHE = {}


def _sc_kernels():
    if _SC_CACHE:
        return _SC_CACHE['gather'], _SC_CACHE['scatter']

    mesh = plsc.VectorSubcoreMesh(core_axis_name="c", subcore_axis_name="s",
                                  num_cores=NC, num_subcores=NS)

    @functools.partial(
        pl.kernel,
        mesh=mesh,
        out_type=[jax.ShapeDtypeStruct((N_EDGES, 32), jnp.float32),
                  jax.ShapeDtypeStruct((N_EDGES, 32), jnp.float32)],
        compiler_params=pltpu.CompilerParams(use_tc_tiling_on_sc=False),
        scratch_types=[pltpu.VMEM((KSUB, CH), jnp.int32),
                       pltpu.VMEM((SCK, 32), jnp.float32),
                       pltpu.SemaphoreType.DMA],
    )
    def sc_gather(pp_hbm, q_hbm, ei_hbm, ts_hbm, td_hbm,
                  idx2_v, rows_v, sem):
        c = lax.axis_index("c")
        s = lax.axis_index("s")
        base = (c * NS + s) * EPW
        brow = (c * NS + s) * NCH

        def one_side(off, row0, side, thbm, ohbm):
            pltpu.sync_copy(ei_hbm.at[side, pl.ds(row0, KSUB)], idx2_v)
            cps = [pltpu.async_copy(
                       thbm.at[idx2_v.at[j]],
                       rows_v.at[pl.ds(j * CH, CH)], sem)
                   for j in range(KSUB)]
            for cp in cps:
                cp.wait()
            pltpu.sync_copy(rows_v, ohbm.at[pl.ds(off, SCK)])

        def body(k, carry):
            off = base + k * SCK
            row0 = brow + k * KSUB
            one_side(off, row0, 0, pp_hbm, ts_hbm)
            one_side(off, row0, 1, q_hbm, td_hbm)
            return carry

        lax.fori_loop(0, NSCK, body, 0)

    @functools.partial(
        pl.kernel,
        mesh=mesh,
        out_type=[jax.ShapeDtypeStruct((NC, NPAD, 32), jnp.float32),
                  jax.ShapeDtypeStruct((NC, NPAD, 32), jnp.float32)],
        compiler_params=pltpu.CompilerParams(use_tc_tiling_on_sc=False),
        scratch_types=[pltpu.VMEM((KSUB, CH), jnp.int32),
                       pltpu.VMEM((SCK, 32), jnp.float32),
                       pltpu.VMEM((CH, 32), jnp.float32),
                       pltpu.VMEM((SL, 32), jnp.float32),
                       pltpu.VMEM_SHARED((NPAD, 32), jnp.float32),
                       pltpu.VMEM_SHARED((NPAD, 32), jnp.float32),
                       pltpu.SemaphoreType.DMA],
    )
    def sc_scatter(e1_hbm, ei_hbm, zeros_hbm, ones_hbm, acc_hbm, cnt_hbm,
                   idx2_v, rows_v, ones_v, stage_v, accS, cntS, sem):
        c = lax.axis_index("c")
        s = lax.axis_index("s")
        base = (c * NS + s) * EPW
        brow = (c * NS + s) * NCH
        pltpu.sync_copy(zeros_hbm, stage_v)
        pltpu.sync_copy(stage_v, accS.at[pl.ds(s * SL, SL)])
        pltpu.sync_copy(stage_v, cntS.at[pl.ds(s * SL, SL)])
        pltpu.sync_copy(ones_hbm, ones_v)
        plsc.subcore_barrier()

        def body(k, carry):
            off = base + k * SCK
            pltpu.sync_copy(ei_hbm.at[0, pl.ds(brow + k * KSUB, KSUB)], idx2_v)
            pltpu.sync_copy(e1_hbm.at[pl.ds(off, SCK)], rows_v)
            cps = []
            for j in range(KSUB):
                cps.append(pltpu.async_copy(
                    rows_v.at[pl.ds(j * CH, CH)],
                    accS.at[idx2_v.at[j]], sem, add=True))
                cps.append(pltpu.async_copy(
                    ones_v, cntS.at[idx2_v.at[j]], sem, add=True))
            for cp in cps:
                cp.wait()
            return carry

        lax.fori_loop(0, NSCK, body, 0)

        plsc.subcore_barrier()
        pltpu.sync_copy(accS.at[pl.ds(s * SL, SL)],
                        acc_hbm.at[c, pl.ds(s * SL, SL)])
        pltpu.sync_copy(cntS.at[pl.ds(s * SL, SL)],
                        cnt_hbm.at[c, pl.ds(s * SL, SL)])

    _SC_CACHE['gather'] = sc_gather
    _SC_CACHE['scatter'] = sc_scatter
    return sc_gather, sc_scatter


@jax.jit
def kernel(x, edge_index, edge_attr, state, batch, params):
    ei3 = edge_index.astype(jnp.int32).reshape(2, N_EDGES // CH, CH)
    batchf = batch.astype(jnp.float32)[None, :]
    pe, pv, pu = params['e'], params['v'], params['u']
    pue, pun, pus = params['ue'], params['un'], params['us']
    w0ue = pue['W0']
    A, Bm, C, D = (w0ue[:, 0:32], w0ue[:, 32:64],
                   w0ue[:, 64:96], w0ue[:, 96:128])

    v, pp, q, u = _pv_call(
        x, batchf, state,
        pu['W0'].T, pu['b0'][None], pu['g0'][None], pu['be0'][None],
        pu['W1'].T, pu['b1'][None], pu['g1'][None], pu['be1'][None], D.T,
        pv['W0'].T, pv['b0'][None], pv['g0'][None], pv['be0'][None],
        pv['W1'].T, pv['b1'][None], pv['g1'][None], pv['be1'][None],
        A.T, Bm.T)

    nE = jnp.float32(N_EDGES)
    eye = jnp.eye(_PK, dtype=jnp.float32)

    def bd(w):
        return jnp.kron(eye, w)

    def t8(b):
        return jnp.tile(b, _PK)[None]

    def fold8(st, d, g, be):
        m = st[0].reshape(_PK, d).sum(0) / nE
        var = st[1].reshape(_PK, d).sum(0) / nE - m * m
        sc = g / jnp.sqrt(var + 1e-5)
        return sc, be - m * sc

    ea_p = edge_attr.reshape(_EP, _PK * 16)
    w0bd = bd(pe['W0'].T)
    b0t = t8(pe['b0'])
    st0 = _pe1_call(ea_p, w0bd, b0t)
    s0, t0 = fold8(st0, 64, pe['g0'], pe['be0'])
    w1fT = (pe['W1'] * s0[None, :]).T
    b1f = pe['b1'] + pe['W1'] @ t0
    h1p, st1 = _pe2_call(ea_p, w0bd, b0t, bd(w1fT), t8(b1f))
    s1, t1 = fold8(st1, 32, pe['g1'], pe['be1'])

    sc_gather, sc_scatter = _sc_kernels()
    ts, td = sc_gather(pp, q, ei3)

    c2T = (C * s1[None, :]).T
    c3 = pue['b0'] + C @ t1
    g0p, st2 = _pe3_call(h1p, ts.reshape(_EP, 256), td.reshape(_EP, 256),
                         bd(c2T), t8(c3))
    s2, t2 = fold8(st2, 32, pue['g0'], pue['be0'])

    w1f2T = (pue['W1'] * s2[None, :]).T
    b1f2 = pue['b1'] + pue['W1'] @ t2
    g1p, st3 = _pe4_call(g0p, bd(w1f2T), t8(b1f2))
    s3, t3 = fold8(st3, 32, pue['g1'], pue['be1'])

    w2fT = (pue['W2'] * s3[None, :]).T
    b2f = pue['b2'] + pue['W2'] @ t3
    g2p, st4 = _pe5_call(g1p, bd(w2fT), t8(b2f))
    s4, t4 = fold8(st4, 32, pue['g2'], pue['be2'])

    e1p = _pe6_call(g2p, h1p, t8(s4), t8(s1), t8(t4 + t1))
    e1 = e1p.reshape(N_EDGES, 32)

    zeros = jnp.zeros((SL, 32), jnp.float32)
    ones = jnp.ones((CH, 32), jnp.float32)
    accp, cntp = sc_scatter(e1, ei3, zeros, ones)

    v1, u1 = _pn_call(
        v, accp, cntp, u, batchf,
        pun['W0'][:, 0:32].T, pun['W0'][:, 32:64].T, pun['W0'][:, 64:96].T,
        pun['b0'][None], pun['g0'][None], pun['be0'][None],
        pun['W1'].T, pun['b1'][None], pun['g1'][None], pun['be1'][None],
        pun['W2'].T, pun['b2'][None], pun['g2'][None], pun['be2'][None],
        pus['W0'][:, 0:32].T, pus['W0'][:, 32:64].T, pus['W0'][:, 64:96].T,
        pus['b0'][None], pus['g0'][None], pus['be0'][None],
        pus['W1'].T, pus['b1'][None], pus['g1'][None], pus['be1'][None],
        pus['W2'].T, pus['b2'][None], pus['g2'][None], pus['be2'][None])

    return (v1, e1, u1)
